# baseline (speedup 1.0000x reference)
import functools

import jax
import jax.numpy as jnp
import numpy as np
from jax import lax
from jax.experimental import pallas as pl
from jax.experimental.pallas import tpu as pltpu
from jax.experimental.pallas import tpu_sc as plsc

_B = 8
_C = 256
_HW = 64
_S8 = 4096
_S4 = 16384
_NQ = 819
_NQP = 832
_SENT = 1e30
_RB = 1024
_NW = 32
_QCH = 208

_pallas_call = pl.pallas_call


def _roll(x, shift):
    return pltpu.roll(x, shift, 1)


def _layernorm(x, g, b, eps=1e-5):
    mu = jnp.mean(x, axis=-1, keepdims=True)
    d = x - mu
    var = jnp.mean(d * d, axis=-1, keepdims=True)
    return d / jnp.sqrt(var + eps) * g + b


def _sigm(x):
    return 1.0 / (1.0 + jnp.exp(-x))


def _const_tables():
    gy, gx = jnp.meshgrid(
        jnp.linspace(0.0, 63.0, 64, dtype=jnp.float32),
        jnp.linspace(0.0, 63.0, 64, dtype=jnp.float32), indexing="ij")
    grid = jnp.stack([gx, gy], axis=-1)
    prop = ((grid + 0.5) / 64.0).reshape(-1, 2)
    valid = jnp.all((prop > 0.01) & (prop < 0.99), axis=-1, keepdims=True)
    op = jnp.log(prop / (1.0 - prop))
    opz = jnp.where(valid, op, 0.0)
    kc = jnp.concatenate(
        [opz, valid.astype(jnp.float32), jnp.zeros((_S8, 1), jnp.float32)],
        axis=1)
    dim_t = jnp.float32(10000.0) ** (
        2.0 * jnp.floor(jnp.arange(128, dtype=jnp.float32) / 2.0) / 128.0)
    even = (jnp.arange(256) % 2 == 0).astype(jnp.float32)
    return kc, dim_t.reshape(1, 128), even.reshape(1, 256)


def _dense_body(mem, kc, eW, eb, ng, nb, cW, cb, W0, b0, W1, b1, W2, b2,
                cls_o, pp_o, cu_o, sc_o):
    x = mem[0]
    k = kc[...]
    validc = k[:, 2:3]
    om0 = jnp.dot(x * validc, eW[...], preferred_element_type=jnp.float32)
    om = _layernorm(om0 + eb[...], ng[...], nb[...])
    cls = jnp.dot(om, cW[...], preferred_element_type=jnp.float32) + cb[...]
    cls_o[0] = cls
    m = jnp.max(cls, axis=-1, keepdims=True)
    e = jnp.exp(cls - m)
    sc_o[0] = e[:, 1:2] / (e[:, 0:1] + e[:, 1:2])
    h = jnp.maximum(
        jnp.dot(om, W0[...], preferred_element_type=jnp.float32) + b0[...], 0.0)
    h = jnp.maximum(
        jnp.dot(h, W1[...], preferred_element_type=jnp.float32) + b1[...], 0.0)
    cur = jnp.dot(h, W2[...], preferred_element_type=jnp.float32) + b2[...] \
        + k[:, 0:2]
    cus = jnp.where(validc > 0.0, cur, _SENT)
    cu_o[0] = cus
    sig = _sigm(cus)
    pp_o[0] = jnp.concatenate([sig[:, 1:2], sig[:, 0:1]], axis=1)


def _dense(mem, kc, eW, eb, ng, nb, cW, cb, W0, b0, W1, b1, W2, b2):
    nrb = _S8 // _RB
    wspec = lambda shp: pl.BlockSpec(shp, lambda b, r: (0,) * len(shp))
    return _pallas_call(
        _dense_body,
        grid=(_B, nrb),
        in_specs=[
            pl.BlockSpec((1, _RB, _C), lambda b, r: (b, _S4 // _RB + r, 0)),
            pl.BlockSpec((_RB, 4), lambda b, r: (r, 0)),
            wspec((_C, _C)), wspec((1, _C)), wspec((1, _C)), wspec((1, _C)),
            wspec((_C, 2)), wspec((1, 2)),
            wspec((_C, _C)), wspec((1, _C)),
            wspec((_C, _C)), wspec((1, _C)),
            wspec((_C, 2)), wspec((1, 2)),
        ],
        out_specs=[
            pl.BlockSpec((1, _RB, 2), lambda b, r: (b, r, 0)),
            pl.BlockSpec((1, _RB, 2), lambda b, r: (b, r, 0)),
            pl.BlockSpec((1, _RB, 2), lambda b, r: (b, r, 0)),
            pl.BlockSpec((1, _RB, 1), lambda b, r: (b, r, 0)),
        ],
        out_shape=[
            jax.ShapeDtypeStruct((_B, _S8, 2), jnp.float32),
            jax.ShapeDtypeStruct((_B, _S8, 2), jnp.float32),
            jax.ShapeDtypeStruct((_B, _S8, 2), jnp.float32),
            jax.ShapeDtypeStruct((_B, _S8, 1), jnp.float32),
        ],
    )(mem, kc, eW, eb, ng, nb, cW, cb, W0, b0, W1, b1, W2, b2)


def _sort_body(sc_ref, idx_o):
    i = lax.broadcasted_iota(jnp.int32, (_B, _S8), 1)
    key = sc_ref[...]
    idx = i
    kk = 2
    while kk <= _S8:
        asc = (i & kk) != 0
        j = kk // 2
        while j >= 1:
            upper = (i & j) != 0
            pkey = jnp.where(upper, _roll(key, j), _roll(key, _S8 - j))
            pidx = jnp.where(upper, _roll(idx, j), _roll(idx, _S8 - j))
            bp = (pkey > key) | ((pkey == key) & (pidx < idx))
            take = bp ^ upper ^ asc
            key = jnp.where(take, pkey, key)
            idx = jnp.where(take, pidx, idx)
            j //= 2
        kk *= 2
    idx_o[...] = idx


def _sort(scores):
    return _pallas_call(
        _sort_body,
        out_shape=jax.ShapeDtypeStruct((_B, _S8), jnp.int32),
    )(scores)


def _rtne(x):
    f = jnp.floor(x)
    d = x - f
    c = f + 1.0
    f_even = jnp.floor(f * 0.5) * 2.0 == f
    return jnp.where(d < 0.5, f, jnp.where(d > 0.5, c, jnp.where(f_even, f, c)))


def _query_body(idx, cu, dvc, ptW, ptb, png, pnb, dimt, even,
                refp_o, qe_o, widx_o, wt_o, didx_o):
    b = pl.program_id(0)
    idxc = idx[0]
    io = lax.broadcasted_iota(jnp.int32, (_NQP, _S8), 1)
    oh = (io == idxc).astype(jnp.float32)
    tc2 = jnp.dot(oh, cu[0], preferred_element_type=jnp.float32,
                  precision=lax.Precision.HIGHEST)
    rx = _sigm(tc2[:, 0:1])
    ry = _sigm(tc2[:, 1:2])
    refp_o[0] = jnp.concatenate([rx, ry], axis=1)
    two_pi = jnp.float32(2.0 * np.pi)
    args = jnp.concatenate(
        [(ry * two_pi) / dimt[...], (rx * two_pi) / dimt[...]], axis=1)
    posq = jnp.where(even[...] > 0.0, jnp.sin(args), jnp.cos(args))
    q0 = jnp.dot(posq, ptW[...], preferred_element_type=jnp.float32) + ptb[...]
    qe_o[0] = _layernorm(q0, png[...], pnb[...])
    gx = 2.0 * rx - 1.0
    gy = 2.0 * ry - 1.0
    ix = (gx + 1.0) * 64.0 / 2.0 - 0.5
    iy = (gy + 1.0) * 64.0 / 2.0 - 0.5
    ix0 = jnp.floor(ix)
    iy0 = jnp.floor(iy)
    ix1 = ix0 + 1.0
    iy1 = iy0 + 1.0
    wx1 = ix - ix0
    wx0 = 1.0 - wx1
    wy1 = iy - iy0
    wy0 = 1.0 - wy1
    boff = b * _S8

    def corner(iyc, ixc, wv):
        ok = (ixc >= 0.0) & (ixc <= 63.0) & (iyc >= 0.0) & (iyc <= 63.0)
        xi = jnp.clip(ixc, 0.0, 63.0).astype(jnp.int32)
        yi = jnp.clip(iyc, 0.0, 63.0).astype(jnp.int32)
        return yi * 64 + xi + boff, jnp.where(ok, wv, 0.0)

    i00, w00 = corner(iy0, ix0, wy0 * wx0)
    i01, w01 = corner(iy0, ix1, wy0 * wx1)
    i10, w10 = corner(iy1, ix0, wy1 * wx0)
    i11, w11 = corner(iy1, ix1, wy1 * wx1)
    widx_o[0] = jnp.concatenate([i00, i01, i10, i11], axis=1)
    wt_o[0] = jnp.concatenate([w00, w01, w10, w11], axis=1)
    rpx = _rtne(rx * 64.0)
    rpy = _rtne(ry * 64.0)
    dvi = jnp.clip(rpy * 64.0 + rpx, 0.0, 4095.0).astype(jnp.int32)
    ohd = (io == dvi).astype(jnp.float32)
    didx_o[0] = jnp.dot(ohd, dvc[0], preferred_element_type=jnp.float32,
                        precision=lax.Precision.HIGHEST)


def _query(idx832, cu, dvc, ptW, ptb, png, pnb, dimt, even):
    wspec = lambda shp: pl.BlockSpec(shp, lambda b: (0,) * len(shp))
    return _pallas_call(
        _query_body,
        grid=(_B,),
        in_specs=[
            pl.BlockSpec((1, _NQP, 1), lambda b: (b, 0, 0)),
            pl.BlockSpec((1, _S8, 2), lambda b: (b, 0, 0)),
            pl.BlockSpec((1, _S8, 1), lambda b: (b, 0, 0)),
            wspec((_C, _C)), wspec((1, _C)), wspec((1, _C)), wspec((1, _C)),
            wspec((1, 128)), wspec((1, 256)),
        ],
        out_specs=[
            pl.BlockSpec((1, _NQP, 2), lambda b: (b, 0, 0)),
            pl.BlockSpec((1, _NQP, _C), lambda b: (b, 0, 0)),
            pl.BlockSpec((1, _NQP, 4), lambda b: (b, 0, 0)),
            pl.BlockSpec((1, _NQP, 4), lambda b: (b, 0, 0)),
            pl.BlockSpec((1, _NQP, 1), lambda b: (b, 0, 0)),
        ],
        out_shape=[
            jax.ShapeDtypeStruct((_B, _NQP, 2), jnp.float32),
            jax.ShapeDtypeStruct((_B, _NQP, _C), jnp.float32),
            jax.ShapeDtypeStruct((_B, _NQP, 4), jnp.int32),
            jax.ShapeDtypeStruct((_B, _NQP, 4), jnp.float32),
            jax.ShapeDtypeStruct((_B, _NQP, 1), jnp.float32),
        ],
    )(idx832, cu, dvc, ptW, ptb, png, pnb, dimt, even)


def _sc_gather(bt, gidx):
    nrows = _B * _NQP * 4
    rpw = nrows // _NW
    nch = rpw // _QCH

    mesh = plsc.VectorSubcoreMesh(core_axis_name="c", subcore_axis_name="s")

    @functools.partial(
        pl.kernel, mesh=mesh,
        out_type=jax.ShapeDtypeStruct((nrows, _C), jnp.float32),
        scratch_types=[
            pltpu.VMEM((_QCH,), jnp.int32),
            pltpu.VMEM((_QCH, _C), jnp.float32),
            pltpu.SemaphoreType.DMA,
        ])
    def body(bt_h, gidx_h, out_h, idx_v, rows_v, sem):
        wid = lax.axis_index("s") * 2 + lax.axis_index("c")
        base = wid * rpw
        for k in range(nch):
            o = base + k * _QCH
            pltpu.sync_copy(gidx_h.at[pl.ds(o, _QCH)], idx_v)
            pltpu.async_copy(bt_h.at[idx_v], rows_v, sem).wait()
            pltpu.sync_copy(rows_v, out_h.at[pl.ds(o, _QCH)])

    return body(bt, gidx)


def _combine_body(co, wt, qf_o):
    cw = co[0] * wt[0]
    qf_o[0] = jnp.sum(cw.reshape(_NQP, 4, _C), axis=1)


def _combine(corners, wts):
    return _pallas_call(
        _combine_body,
        grid=(_B,),
        in_specs=[
            pl.BlockSpec((1, _NQP * 4, _C), lambda b: (b, 0, 0)),
            pl.BlockSpec((1, _NQP * 4, 1), lambda b: (b, 0, 0)),
        ],
        out_specs=pl.BlockSpec((1, _NQP, _C), lambda b: (b, 0, 0)),
        out_shape=jax.ShapeDtypeStruct((_B, _NQP, _C), jnp.float32),
    )(corners, wts)


def kernel(memory, mask_flatten, backbone_features, div,
           enc_output_W, enc_output_b, enc_norm_g, enc_norm_b,
           cls_W, cls_b, mlp_W0, mlp_b0, mlp_W1, mlp_b1, mlp_W2, mlp_b2,
           pt_W, pt_b, ptn_g, ptn_b):
    f32 = jnp.float32
    kc, dimt, even = _const_tables()
    r1 = lambda v: v.reshape(1, -1).astype(f32)
    cls_out, pp, cu, sc3 = _dense(
        memory, kc, enc_output_W, r1(enc_output_b), r1(enc_norm_g),
        r1(enc_norm_b), cls_W, r1(cls_b), mlp_W0, r1(mlp_b0),
        mlp_W1, r1(mlp_b1), mlp_W2, r1(mlp_b2))
    sidx = _sort(sc3.reshape(_B, _S8))
    idx832 = sidx[:, :_NQP].reshape(_B, _NQP, 1)
    refp, qe, widx, wt, dnew = _query(
        idx832, cu, div.reshape(_B, _S8, 1), pt_W, r1(pt_b), r1(ptn_g),
        r1(ptn_b), dimt, even)
    bt = backbone_features.reshape(_B, _C, _S8).transpose(0, 2, 1) \
        .reshape(_B * _S8, _C)
    corners = _sc_gather(bt, widx.reshape(-1))
    qf = _combine(corners.reshape(_B, _NQP * 4, _C),
                  wt.reshape(_B, _NQP * 4, 1))
    return (cls_out, pp, dnew.reshape(_B, _NQP)[:, :_NQ],
            qf[:, :_NQ], qe[:, :_NQ], refp[:, :_NQ])


# speedup vs baseline: 1.6572x; 1.6572x over previous
import functools

import jax
import jax.numpy as jnp
import numpy as np
from jax import lax
from jax.experimental import pallas as pl
from jax.experimental.pallas import tpu as pltpu
from jax.experimental.pallas import tpu_sc as plsc

_B = 8
_C = 256
_HW = 64
_S8 = 4096
_S4 = 16384
_NQ = 819
_NQP = 832
_SENT = 1e30
_RB = 1024
_NW = 32
_QCH = 208

_pallas_call = pl.pallas_call


def _roll(x, shift):
    return pltpu.roll(x, shift, 1)


def _layernorm(x, g, b, eps=1e-5):
    mu = jnp.mean(x, axis=-1, keepdims=True)
    d = x - mu
    var = jnp.mean(d * d, axis=-1, keepdims=True)
    return d / jnp.sqrt(var + eps) * g + b


def _sigm(x):
    return 1.0 / (1.0 + jnp.exp(-x))


def _const_tables():
    gy, gx = jnp.meshgrid(
        jnp.linspace(0.0, 63.0, 64, dtype=jnp.float32),
        jnp.linspace(0.0, 63.0, 64, dtype=jnp.float32), indexing="ij")
    grid = jnp.stack([gx, gy], axis=-1)
    prop = ((grid + 0.5) / 64.0).reshape(-1, 2)
    valid = jnp.all((prop > 0.01) & (prop < 0.99), axis=-1, keepdims=True)
    op = jnp.log(prop / (1.0 - prop))
    opz = jnp.where(valid, op, 0.0)
    kc = jnp.concatenate(
        [opz, valid.astype(jnp.float32), jnp.zeros((_S8, 1), jnp.float32)],
        axis=1)
    dim_t = jnp.float32(10000.0) ** (
        2.0 * jnp.floor(jnp.arange(128, dtype=jnp.float32) / 2.0) / 128.0)
    even = (jnp.arange(256) % 2 == 0).astype(jnp.float32)
    return kc, dim_t.reshape(1, 128), even.reshape(1, 256)


def _dense_body(mem, kc, eW, eb, ng, nb, cW, cb, W0, b0, W1, b1, W2, b2,
                cls_o, pp_o, cu_o, sc_o):
    x = mem[0]
    k = kc[...]
    validc = k[:, 2:3]
    om0 = jnp.dot(x * validc, eW[...], preferred_element_type=jnp.float32)
    om = _layernorm(om0 + eb[...], ng[...], nb[...])
    cls = jnp.dot(om, cW[...], preferred_element_type=jnp.float32) + cb[...]
    cls_o[0] = cls
    m = jnp.max(cls, axis=-1, keepdims=True)
    e = jnp.exp(cls - m)
    sc_o[0] = e[:, 1:2] / (e[:, 0:1] + e[:, 1:2])
    h = jnp.maximum(
        jnp.dot(om, W0[...], preferred_element_type=jnp.float32) + b0[...], 0.0)
    h = jnp.maximum(
        jnp.dot(h, W1[...], preferred_element_type=jnp.float32) + b1[...], 0.0)
    cur = jnp.dot(h, W2[...], preferred_element_type=jnp.float32) + b2[...] \
        + k[:, 0:2]
    cus = jnp.where(validc > 0.0, cur, _SENT)
    cu_o[0] = cus
    sig = _sigm(cus)
    pp_o[0] = jnp.concatenate([sig[:, 1:2], sig[:, 0:1]], axis=1)


def _dense(mem, kc, eW, eb, ng, nb, cW, cb, W0, b0, W1, b1, W2, b2):
    nrb = _S8 // _RB
    wspec = lambda shp: pl.BlockSpec(shp, lambda b, r: (0,) * len(shp))
    return _pallas_call(
        _dense_body,
        grid=(_B, nrb),
        in_specs=[
            pl.BlockSpec((1, _RB, _C), lambda b, r: (b, _S4 // _RB + r, 0)),
            pl.BlockSpec((_RB, 4), lambda b, r: (r, 0)),
            wspec((_C, _C)), wspec((1, _C)), wspec((1, _C)), wspec((1, _C)),
            wspec((_C, 2)), wspec((1, 2)),
            wspec((_C, _C)), wspec((1, _C)),
            wspec((_C, _C)), wspec((1, _C)),
            wspec((_C, 2)), wspec((1, 2)),
        ],
        out_specs=[
            pl.BlockSpec((1, _RB, 2), lambda b, r: (b, r, 0)),
            pl.BlockSpec((1, _RB, 2), lambda b, r: (b, r, 0)),
            pl.BlockSpec((1, _RB, 2), lambda b, r: (b, r, 0)),
            pl.BlockSpec((1, _RB, 1), lambda b, r: (b, r, 0)),
        ],
        out_shape=[
            jax.ShapeDtypeStruct((_B, _S8, 2), jnp.float32),
            jax.ShapeDtypeStruct((_B, _S8, 2), jnp.float32),
            jax.ShapeDtypeStruct((_B, _S8, 2), jnp.float32),
            jax.ShapeDtypeStruct((_B, _S8, 1), jnp.float32),
        ],
    )(mem, kc, eW, eb, ng, nb, cW, cb, W0, b0, W1, b1, W2, b2)


def _sort_body(sc_ref, idx_o):
    i = lax.broadcasted_iota(jnp.int32, (_B, _S8), 1)
    key = sc_ref[...]
    idx = i
    kk = 2
    while kk <= _S8:
        asc = (i & kk) != 0
        j = kk // 2
        while j >= 1:
            upper = (i & j) != 0
            pkey = jnp.where(upper, _roll(key, j), _roll(key, _S8 - j))
            pidx = jnp.where(upper, _roll(idx, j), _roll(idx, _S8 - j))
            bp = (pkey > key) | ((pkey == key) & (pidx < idx))
            take = bp ^ upper ^ asc
            key = jnp.where(take, pkey, key)
            idx = jnp.where(take, pidx, idx)
            j //= 2
        kk *= 2
    idx_o[...] = idx


def _sort(scores):
    return _pallas_call(
        _sort_body,
        out_shape=jax.ShapeDtypeStruct((_B, _S8), jnp.int32),
    )(scores)


def _rtne(x):
    f = jnp.floor(x)
    d = x - f
    c = f + 1.0
    f_even = jnp.floor(f * 0.5) * 2.0 == f
    return jnp.where(d < 0.5, f, jnp.where(d > 0.5, c, jnp.where(f_even, f, c)))


def _query_body(idx, cu, dvc, ptW, ptb, png, pnb, dimt, even,
                refp_o, qe_o, widx_o, wt_o, didx_o):
    b = pl.program_id(0)
    idxr = idx[0]
    io = lax.broadcasted_iota(jnp.int32, (_S8, _NQP), 0)
    oh = (io == idxr).astype(jnp.float32)
    tc2 = jnp.dot(cu[0], oh, preferred_element_type=jnp.float32,
                  precision=lax.Precision.HIGHEST)
    rx = _sigm(tc2[0:1, :])
    ry = _sigm(tc2[1:2, :])
    refp_o[0] = jnp.concatenate([rx, ry], axis=0)
    two_pi = jnp.float32(2.0 * np.pi)
    args = jnp.concatenate(
        [(ry * two_pi) / dimt[...], (rx * two_pi) / dimt[...]], axis=0)
    posq = jnp.where(even[...] > 0.0, jnp.sin(args), jnp.cos(args))
    q0 = jnp.dot(ptW[...], posq, preferred_element_type=jnp.float32) \
        + ptb[...]
    mu = jnp.mean(q0, axis=0, keepdims=True)
    dd = q0 - mu
    var = jnp.mean(dd * dd, axis=0, keepdims=True)
    qe_o[0] = dd / jnp.sqrt(var + 1e-5) * png[...] + pnb[...]
    gx = 2.0 * rx - 1.0
    gy = 2.0 * ry - 1.0
    ix = (gx + 1.0) * 64.0 / 2.0 - 0.5
    iy = (gy + 1.0) * 64.0 / 2.0 - 0.5
    ix0 = jnp.floor(ix)
    iy0 = jnp.floor(iy)
    ix1 = ix0 + 1.0
    iy1 = iy0 + 1.0
    wx1 = ix - ix0
    wx0 = 1.0 - wx1
    wy1 = iy - iy0
    wy0 = 1.0 - wy1
    boff = b * _S8

    def corner(iyc, ixc, wv):
        ok = (ixc >= 0.0) & (ixc <= 63.0) & (iyc >= 0.0) & (iyc <= 63.0)
        xi = jnp.clip(ixc, 0.0, 63.0).astype(jnp.int32)
        yi = jnp.clip(iyc, 0.0, 63.0).astype(jnp.int32)
        return yi * 64 + xi + boff, jnp.where(ok, wv, 0.0)

    i00, w00 = corner(iy0, ix0, wy0 * wx0)
    i01, w01 = corner(iy0, ix1, wy0 * wx1)
    i10, w10 = corner(iy1, ix0, wy1 * wx0)
    i11, w11 = corner(iy1, ix1, wy1 * wx1)
    widx_o[0] = jnp.concatenate([i00, i01, i10, i11], axis=0)
    wt_o[0] = jnp.concatenate([w00, w01, w10, w11], axis=0)
    rpx = _rtne(rx * 64.0)
    rpy = _rtne(ry * 64.0)
    dvi = jnp.clip(rpy * 64.0 + rpx, 0.0, 4095.0).astype(jnp.int32)
    ohd = (io == dvi).astype(jnp.float32)
    didx_o[0] = jnp.dot(dvc[0], ohd, preferred_element_type=jnp.float32,
                        precision=lax.Precision.HIGHEST)


def _query(idx832, cu, dvc, ptW, ptb, png, pnb, dimt, even):
    wspec = lambda shp: pl.BlockSpec(shp, lambda b: (0,) * len(shp))
    return _pallas_call(
        _query_body,
        grid=(_B,),
        in_specs=[
            pl.BlockSpec((1, 1, _NQP), lambda b: (b, 0, 0)),
            pl.BlockSpec((1, 2, _S8), lambda b: (b, 0, 0)),
            pl.BlockSpec((1, 1, _S8), lambda b: (b, 0, 0)),
            wspec((_C, _C)), wspec((_C, 1)), wspec((_C, 1)), wspec((_C, 1)),
            wspec((128, 1)), wspec((_C, 1)),
        ],
        out_specs=[
            pl.BlockSpec((1, 2, _NQP), lambda b: (b, 0, 0)),
            pl.BlockSpec((1, _C, _NQP), lambda b: (b, 0, 0)),
            pl.BlockSpec((1, 4, _NQP), lambda b: (b, 0, 0)),
            pl.BlockSpec((1, 4, _NQP), lambda b: (b, 0, 0)),
            pl.BlockSpec((1, 1, _NQP), lambda b: (b, 0, 0)),
        ],
        out_shape=[
            jax.ShapeDtypeStruct((_B, 2, _NQP), jnp.float32),
            jax.ShapeDtypeStruct((_B, _C, _NQP), jnp.float32),
            jax.ShapeDtypeStruct((_B, 4, _NQP), jnp.int32),
            jax.ShapeDtypeStruct((_B, 4, _NQP), jnp.float32),
            jax.ShapeDtypeStruct((_B, 1, _NQP), jnp.float32),
        ],
    )(idx832, cu, dvc, ptW, ptb, png, pnb, dimt, even)


def _sc_gather(bt, gidx):
    nrows = _B * _NQP * 4
    rpw = nrows // _NW
    nch = rpw // _QCH

    mesh = plsc.VectorSubcoreMesh(core_axis_name="c", subcore_axis_name="s")

    @functools.partial(
        pl.kernel, mesh=mesh,
        out_type=jax.ShapeDtypeStruct((nrows, _C), jnp.float32),
        scratch_types=[
            pltpu.VMEM((_QCH,), jnp.int32),
            pltpu.VMEM((_QCH, _C), jnp.float32),
            pltpu.SemaphoreType.DMA,
        ])
    def body(bt_h, gidx_h, out_h, idx_v, rows_v, sem):
        wid = lax.axis_index("s") * 2 + lax.axis_index("c")
        base = wid * rpw
        for k in range(nch):
            o = base + k * _QCH
            pltpu.sync_copy(gidx_h.at[pl.ds(o, _QCH)], idx_v)
            pltpu.async_copy(bt_h.at[idx_v], rows_v, sem).wait()
            pltpu.sync_copy(rows_v, out_h.at[pl.ds(o, _QCH)])

    return body(bt, gidx)


def _combine_body(co, wt, qf_o):
    cw = co[0] * wt[0]
    qf_o[0] = jnp.sum(cw.reshape(_NQP, 4, _C), axis=1)


def _combine(corners, wts):
    return _pallas_call(
        _combine_body,
        grid=(_B,),
        in_specs=[
            pl.BlockSpec((1, _NQP * 4, _C), lambda b: (b, 0, 0)),
            pl.BlockSpec((1, _NQP * 4, 1), lambda b: (b, 0, 0)),
        ],
        out_specs=pl.BlockSpec((1, _NQP, _C), lambda b: (b, 0, 0)),
        out_shape=jax.ShapeDtypeStruct((_B, _NQP, _C), jnp.float32),
    )(corners, wts)


def kernel(memory, mask_flatten, backbone_features, div,
           enc_output_W, enc_output_b, enc_norm_g, enc_norm_b,
           cls_W, cls_b, mlp_W0, mlp_b0, mlp_W1, mlp_b1, mlp_W2, mlp_b2,
           pt_W, pt_b, ptn_g, ptn_b):
    f32 = jnp.float32
    kc, dimt, even = _const_tables()
    r1 = lambda v: v.reshape(1, -1).astype(f32)
    cls_out, pp, cu, sc3 = _dense(
        memory, kc, enc_output_W, r1(enc_output_b), r1(enc_norm_g),
        r1(enc_norm_b), cls_W, r1(cls_b), mlp_W0, r1(mlp_b0),
        mlp_W1, r1(mlp_b1), mlp_W2, r1(mlp_b2))
    sidx = _sort(sc3.reshape(_B, _S8))
    idx832 = sidx[:, :_NQP].reshape(_B, 1, _NQP)
    rc = lambda v: v.reshape(-1, 1).astype(f32)
    refpT, qeT, widxT, wtT, dnew = _query(
        idx832, cu.transpose(0, 2, 1), div.reshape(_B, 1, _S8),
        pt_W.T, rc(pt_b), rc(ptn_g), rc(ptn_b),
        dimt.reshape(128, 1), even.reshape(_C, 1))
    bt = backbone_features.reshape(_B, _C, _S8).transpose(0, 2, 1) \
        .reshape(_B * _S8, _C)
    corners = _sc_gather(bt, widxT.transpose(0, 2, 1).reshape(-1))
    qf = _combine(corners.reshape(_B, _NQP * 4, _C),
                  wtT.transpose(0, 2, 1).reshape(_B, _NQP * 4, 1))
    return (cls_out, pp, dnew.reshape(_B, _NQP)[:, :_NQ],
            qf[:, :_NQ], qeT.transpose(0, 2, 1)[:, :_NQ],
            refpT.transpose(0, 2, 1)[:, :_NQ])


# speedup vs baseline: 1.9731x; 1.1906x over previous
import functools

import jax
import jax.numpy as jnp
import numpy as np
from jax import lax
from jax.experimental import pallas as pl
from jax.experimental.pallas import tpu as pltpu
from jax.experimental.pallas import tpu_sc as plsc

_B = 8
_C = 256
_HW = 64
_S8 = 4096
_S4 = 16384
_NQ = 819
_NQP = 832
_SENT = 1e30
_RB = 1024
_NW = 32
_QCH = 208

_pallas_call = pl.pallas_call


def _roll(x, shift):
    return pltpu.roll(x, shift, 1)


def _layernorm(x, g, b, eps=1e-5):
    mu = jnp.mean(x, axis=-1, keepdims=True)
    d = x - mu
    var = jnp.mean(d * d, axis=-1, keepdims=True)
    return d / jnp.sqrt(var + eps) * g + b


def _sigm(x):
    return 1.0 / (1.0 + jnp.exp(-x))


def _const_tables():
    gy, gx = jnp.meshgrid(
        jnp.linspace(0.0, 63.0, 64, dtype=jnp.float32),
        jnp.linspace(0.0, 63.0, 64, dtype=jnp.float32), indexing="ij")
    grid = jnp.stack([gx, gy], axis=-1)
    prop = ((grid + 0.5) / 64.0).reshape(-1, 2)
    valid = jnp.all((prop > 0.01) & (prop < 0.99), axis=-1, keepdims=True)
    op = jnp.log(prop / (1.0 - prop))
    opz = jnp.where(valid, op, 0.0)
    kc = jnp.concatenate(
        [opz, valid.astype(jnp.float32), jnp.zeros((_S8, 1), jnp.float32)],
        axis=1)
    dim_t = jnp.float32(10000.0) ** (
        2.0 * jnp.floor(jnp.arange(128, dtype=jnp.float32) / 2.0) / 128.0)
    even = (jnp.arange(256) % 2 == 0).astype(jnp.float32)
    return kc, dim_t.reshape(1, 128), even.reshape(1, 256)


def _dense_body(mem, kc, dv, eW, eb, ng, nb, cW, cb, W0, b0, W1, b1, W2, b2,
                cls_o, pp_o, gt_o, sc_o):
    x = mem[0]
    k = kc[...]
    validc = k[:, 2:3]
    om0 = jnp.dot(x * validc, eW[...], preferred_element_type=jnp.float32)
    om = _layernorm(om0 + eb[...], ng[...], nb[...])
    cls = jnp.dot(om, cW[...], preferred_element_type=jnp.float32) + cb[...]
    cls_o[0] = cls
    m = jnp.max(cls, axis=-1, keepdims=True)
    e = jnp.exp(cls - m)
    sc_o[0] = e[:, 1:2] / (e[:, 0:1] + e[:, 1:2])
    h = jnp.maximum(
        jnp.dot(om, W0[...], preferred_element_type=jnp.float32) + b0[...], 0.0)
    h = jnp.maximum(
        jnp.dot(h, W1[...], preferred_element_type=jnp.float32) + b1[...], 0.0)
    cur = jnp.dot(h, W2[...], preferred_element_type=jnp.float32) + b2[...] \
        + k[:, 0:2]
    cus = jnp.where(validc > 0.0, cur, _SENT)
    gt_o[0] = jnp.concatenate(
        [cus, dv[0], jnp.zeros((_RB, 125), jnp.float32)], axis=1)
    sig = _sigm(cus)
    pp_o[0] = jnp.concatenate([sig[:, 1:2], sig[:, 0:1]], axis=1)


def _dense(mem, kc, dvr, eW, eb, ng, nb, cW, cb, W0, b0, W1, b1, W2, b2):
    nrb = _S8 // _RB
    wspec = lambda shp: pl.BlockSpec(shp, lambda b, r: (0,) * len(shp))
    return _pallas_call(
        _dense_body,
        grid=(_B, nrb),
        in_specs=[
            pl.BlockSpec((1, _RB, _C), lambda b, r: (b, _S4 // _RB + r, 0)),
            pl.BlockSpec((_RB, 4), lambda b, r: (r, 0)),
            pl.BlockSpec((1, _RB, 1), lambda b, r: (b, r, 0)),
            wspec((_C, _C)), wspec((1, _C)), wspec((1, _C)), wspec((1, _C)),
            wspec((_C, 2)), wspec((1, 2)),
            wspec((_C, _C)), wspec((1, _C)),
            wspec((_C, _C)), wspec((1, _C)),
            wspec((_C, 2)), wspec((1, 2)),
        ],
        out_specs=[
            pl.BlockSpec((1, _RB, 2), lambda b, r: (b, r, 0)),
            pl.BlockSpec((1, _RB, 2), lambda b, r: (b, r, 0)),
            pl.BlockSpec((1, _RB, 128), lambda b, r: (b, r, 0)),
            pl.BlockSpec((1, _RB, 1), lambda b, r: (b, r, 0)),
        ],
        out_shape=[
            jax.ShapeDtypeStruct((_B, _S8, 2), jnp.float32),
            jax.ShapeDtypeStruct((_B, _S8, 2), jnp.float32),
            jax.ShapeDtypeStruct((_B, _S8, 128), jnp.float32),
            jax.ShapeDtypeStruct((_B, _S8, 1), jnp.float32),
        ],
    )(mem, kc, dvr, eW, eb, ng, nb, cW, cb, W0, b0, W1, b1, W2, b2)


def _sort_body(sc_ref, idx_o):
    i = lax.broadcasted_iota(jnp.int32, (_B, _S8), 1)
    key = sc_ref[...]
    idx = i
    kk = 2
    while kk <= _S8:
        asc = (i & kk) != 0
        j = kk // 2
        while j >= 1:
            upper = (i & j) != 0
            pkey = jnp.where(upper, _roll(key, j), _roll(key, _S8 - j))
            pidx = jnp.where(upper, _roll(idx, j), _roll(idx, _S8 - j))
            bp = (pkey > key) | ((pkey == key) & (pidx < idx))
            take = bp ^ upper ^ asc
            key = jnp.where(take, pkey, key)
            idx = jnp.where(take, pidx, idx)
            j //= 2
        kk *= 2
    idx_o[...] = idx


def _sort(scores):
    return _pallas_call(
        _sort_body,
        out_shape=jax.ShapeDtypeStruct((_B, _S8), jnp.int32),
    )(scores)


def _rtne(x):
    f = jnp.floor(x)
    d = x - f
    c = f + 1.0
    f_even = jnp.floor(f * 0.5) * 2.0 == f
    return jnp.where(d < 0.5, f, jnp.where(d > 0.5, c, jnp.where(f_even, f, c)))


def _query_body(qr, ptW, ptb, png, pnb, dimt, even,
                refp_o, qe_o, widx_o, wt_o, didx_o):
    b = pl.program_id(0)
    rows = qr[0]
    rx = _sigm(rows[:, 0:1])
    ry = _sigm(rows[:, 1:2])
    refp_o[0] = jnp.concatenate([rx, ry], axis=1)
    two_pi = jnp.float32(2.0 * np.pi)
    args = jnp.concatenate(
        [(ry * two_pi) / dimt[...], (rx * two_pi) / dimt[...]], axis=1)
    posq = jnp.where(even[...] > 0.0, jnp.sin(args), jnp.cos(args))
    q0 = jnp.dot(posq, ptW[...], preferred_element_type=jnp.float32) \
        + ptb[...]
    qe_o[0] = _layernorm(q0, png[...], pnb[...])
    gx = 2.0 * rx - 1.0
    gy = 2.0 * ry - 1.0
    ix = (gx + 1.0) * 64.0 / 2.0 - 0.5
    iy = (gy + 1.0) * 64.0 / 2.0 - 0.5
    ix0 = jnp.floor(ix)
    iy0 = jnp.floor(iy)
    ix1 = ix0 + 1.0
    iy1 = iy0 + 1.0
    wx1 = ix - ix0
    wx0 = 1.0 - wx1
    wy1 = iy - iy0
    wy0 = 1.0 - wy1
    boff = b * _S8

    def corner(iyc, ixc, wv):
        ok = (ixc >= 0.0) & (ixc <= 63.0) & (iyc >= 0.0) & (iyc <= 63.0)
        xi = jnp.clip(ixc, 0.0, 63.0).astype(jnp.int32)
        yi = jnp.clip(iyc, 0.0, 63.0).astype(jnp.int32)
        return yi * 64 + xi + boff, jnp.where(ok, wv, 0.0)

    i00, w00 = corner(iy0, ix0, wy0 * wx0)
    i01, w01 = corner(iy0, ix1, wy0 * wx1)
    i10, w10 = corner(iy1, ix0, wy1 * wx0)
    i11, w11 = corner(iy1, ix1, wy1 * wx1)
    widx_o[0] = jnp.concatenate([i00, i01, i10, i11], axis=1)
    wt_o[0] = jnp.concatenate([w00, w01, w10, w11], axis=1)
    rpx = _rtne(rx * 64.0)
    rpy = _rtne(ry * 64.0)
    dvi = jnp.clip(rpy * 64.0 + rpx, 0.0, 4095.0).astype(jnp.int32)
    didx_o[0] = dvi + boff


def _query(qrows, ptW, ptb, png, pnb, dimt, even):
    wspec = lambda shp: pl.BlockSpec(shp, lambda b: (0,) * len(shp))
    return _pallas_call(
        _query_body,
        grid=(_B,),
        in_specs=[
            pl.BlockSpec((1, _NQP, 128), lambda b: (b, 0, 0)),
            wspec((_C, _C)), wspec((1, _C)), wspec((1, _C)), wspec((1, _C)),
            wspec((1, 128)), wspec((1, _C)),
        ],
        out_specs=[
            pl.BlockSpec((1, _NQP, 2), lambda b: (b, 0, 0)),
            pl.BlockSpec((1, _NQP, _C), lambda b: (b, 0, 0)),
            pl.BlockSpec((1, _NQP, 4), lambda b: (b, 0, 0)),
            pl.BlockSpec((1, _NQP, 4), lambda b: (b, 0, 0)),
            pl.BlockSpec((1, _NQP, 1), lambda b: (b, 0, 0)),
        ],
        out_shape=[
            jax.ShapeDtypeStruct((_B, _NQP, 2), jnp.float32),
            jax.ShapeDtypeStruct((_B, _NQP, _C), jnp.float32),
            jax.ShapeDtypeStruct((_B, _NQP, 4), jnp.int32),
            jax.ShapeDtypeStruct((_B, _NQP, 4), jnp.float32),
            jax.ShapeDtypeStruct((_B, _NQP, 1), jnp.int32),
        ],
    )(qrows, ptW, ptb, png, pnb, dimt, even)


def _sc_gather1(gtab, qidx):
    ndiv = _B * _NQP
    dpw = ndiv // _NW

    mesh = plsc.VectorSubcoreMesh(core_axis_name="c", subcore_axis_name="s")

    @functools.partial(
        pl.kernel, mesh=mesh,
        out_type=jax.ShapeDtypeStruct((ndiv, 128), jnp.float32),
        scratch_types=[
            pltpu.VMEM((dpw,), jnp.int32),
            pltpu.VMEM((dpw, 128), jnp.float32),
            pltpu.SemaphoreType.DMA,
        ])
    def body(gt_h, qidx_h, out_h, idx_v, rows_v, sem):
        wid = lax.axis_index("s") * 2 + lax.axis_index("c")
        base = wid * dpw
        pltpu.sync_copy(qidx_h.at[pl.ds(base, dpw)], idx_v)
        pltpu.async_copy(gt_h.at[idx_v], rows_v, sem).wait()
        pltpu.sync_copy(rows_v, out_h.at[pl.ds(base, dpw)])

    return body(gtab, qidx)


def _sc_gather(bt, gidx, gtab, dgidx):
    nrows = _B * _NQP * 4
    rpw = nrows // _NW
    nch = rpw // _QCH
    ndiv = _B * _NQP
    dpw = ndiv // _NW

    mesh = plsc.VectorSubcoreMesh(core_axis_name="c", subcore_axis_name="s")

    @functools.partial(
        pl.kernel, mesh=mesh,
        out_type=(jax.ShapeDtypeStruct((nrows, _C), jnp.float32),
                  jax.ShapeDtypeStruct((ndiv, 128), jnp.float32)),
        scratch_types=[
            pltpu.VMEM((_QCH,), jnp.int32),
            pltpu.VMEM((_QCH, _C), jnp.float32),
            pltpu.VMEM((dpw,), jnp.int32),
            pltpu.VMEM((dpw, 128), jnp.float32),
            pltpu.SemaphoreType.DMA,
        ])
    def body(bt_h, gidx_h, gt_h, dgidx_h, out_h, dnr_h,
             idx_v, rows_v, didx_v, drow_v, sem):
        wid = lax.axis_index("s") * 2 + lax.axis_index("c")
        base = wid * rpw
        for k in range(nch):
            o = base + k * _QCH
            pltpu.sync_copy(gidx_h.at[pl.ds(o, _QCH)], idx_v)
            pltpu.async_copy(bt_h.at[idx_v], rows_v, sem).wait()
            pltpu.sync_copy(rows_v, out_h.at[pl.ds(o, _QCH)])
        dbase = wid * dpw
        pltpu.sync_copy(dgidx_h.at[pl.ds(dbase, dpw)], didx_v)
        pltpu.async_copy(gt_h.at[didx_v], drow_v, sem).wait()
        pltpu.sync_copy(drow_v, dnr_h.at[pl.ds(dbase, dpw)])

    return body(bt, gidx, gtab, dgidx)


def _combine_body(co, wt, qf_o):
    cw = co[0] * wt[0]
    qf_o[0] = jnp.sum(cw.reshape(_NQP, 4, _C), axis=1)


def _combine(corners, wts):
    return _pallas_call(
        _combine_body,
        grid=(_B,),
        in_specs=[
            pl.BlockSpec((1, _NQP * 4, _C), lambda b: (b, 0, 0)),
            pl.BlockSpec((1, _NQP * 4, 1), lambda b: (b, 0, 0)),
        ],
        out_specs=pl.BlockSpec((1, _NQP, _C), lambda b: (b, 0, 0)),
        out_shape=jax.ShapeDtypeStruct((_B, _NQP, _C), jnp.float32),
    )(corners, wts)


def kernel(memory, mask_flatten, backbone_features, div,
           enc_output_W, enc_output_b, enc_norm_g, enc_norm_b,
           cls_W, cls_b, mlp_W0, mlp_b0, mlp_W1, mlp_b1, mlp_W2, mlp_b2,
           pt_W, pt_b, ptn_g, ptn_b):
    f32 = jnp.float32
    kc, dimt, even = _const_tables()
    r1 = lambda v: v.reshape(1, -1).astype(f32)
    cls_out, pp, gtab, sc3 = _dense(
        memory, kc, div.reshape(_B, _S8, 1), enc_output_W, r1(enc_output_b),
        r1(enc_norm_g), r1(enc_norm_b), cls_W, r1(cls_b), mlp_W0, r1(mlp_b0),
        mlp_W1, r1(mlp_b1), mlp_W2, r1(mlp_b2))
    sidx = _sort(sc3.reshape(_B, _S8))
    gt_flat = gtab.reshape(_B * _S8, 128)
    qidx = (sidx[:, :_NQP]
            + (jnp.arange(_B, dtype=jnp.int32) * _S8)[:, None]).reshape(-1)
    qrows = _sc_gather1(gt_flat, qidx)
    refp, qe, widx, wt, didx = _query(
        qrows.reshape(_B, _NQP, 128), pt_W, r1(pt_b), r1(ptn_g), r1(ptn_b),
        dimt, even)
    bt = backbone_features.reshape(_B, _C, _S8).transpose(0, 2, 1) \
        .reshape(_B * _S8, _C)
    corners, dnr = _sc_gather(bt, widx.reshape(-1), gt_flat,
                              didx.reshape(-1))
    qf = _combine(corners.reshape(_B, _NQP * 4, _C),
                  wt.reshape(_B, _NQP * 4, 1))
    return (cls_out, pp, dnr[:, 2].reshape(_B, _NQP)[:, :_NQ],
            qf[:, :_NQ], qe[:, :_NQ], refp[:, :_NQ])


# speedup vs baseline: 2.0756x; 1.0520x over previous
import functools

import jax
import jax.numpy as jnp
import numpy as np
from jax import lax
from jax.experimental import pallas as pl
from jax.experimental.pallas import tpu as pltpu
from jax.experimental.pallas import tpu_sc as plsc

_B = 8
_C = 256
_HW = 64
_S8 = 4096
_S4 = 16384
_NQ = 819
_NQP = 832
_SENT = 1e30
_RB = 1024
_NW = 32
_QCH = 208

_pallas_call = pl.pallas_call


def _roll(x, shift):
    return pltpu.roll(x, shift, 1)


def _layernorm(x, g, b, eps=1e-5):
    mu = jnp.mean(x, axis=-1, keepdims=True)
    d = x - mu
    var = jnp.mean(d * d, axis=-1, keepdims=True)
    return d / jnp.sqrt(var + eps) * g + b


def _sigm(x):
    return 1.0 / (1.0 + jnp.exp(-x))


def _const_tables():
    gy, gx = jnp.meshgrid(
        jnp.linspace(0.0, 63.0, 64, dtype=jnp.float32),
        jnp.linspace(0.0, 63.0, 64, dtype=jnp.float32), indexing="ij")
    grid = jnp.stack([gx, gy], axis=-1)
    prop = ((grid + 0.5) / 64.0).reshape(-1, 2)
    valid = jnp.all((prop > 0.01) & (prop < 0.99), axis=-1, keepdims=True)
    op = jnp.log(prop / (1.0 - prop))
    opz = jnp.where(valid, op, 0.0)
    kc = jnp.concatenate(
        [opz, valid.astype(jnp.float32), jnp.zeros((_S8, 1), jnp.float32)],
        axis=1)
    dim_t = jnp.float32(10000.0) ** (
        2.0 * jnp.floor(jnp.arange(128, dtype=jnp.float32) / 2.0) / 128.0)
    even = (jnp.arange(256) % 2 == 0).astype(jnp.float32)
    return kc, dim_t.reshape(1, 128), even.reshape(1, 256)


def _dense_body(mem, kc, dv, eW, eb, ng, nb, cW, cb, W0, b0, W1, b1, W2, b2,
                cls_o, pp_o, gt_o, sc_o):
    x = mem[0]
    k = kc[...]
    validc = k[:, 2:3]
    om0 = jnp.dot(x * validc, eW[...], preferred_element_type=jnp.float32)
    om = _layernorm(om0 + eb[...], ng[...], nb[...])
    cls = jnp.dot(om, cW[...], preferred_element_type=jnp.float32) + cb[...]
    cls_o[0] = cls
    m = jnp.max(cls, axis=-1, keepdims=True)
    e = jnp.exp(cls - m)
    sc_o[0] = e[:, 1:2] / (e[:, 0:1] + e[:, 1:2])
    h = jnp.maximum(
        jnp.dot(om, W0[...], preferred_element_type=jnp.float32) + b0[...], 0.0)
    h = jnp.maximum(
        jnp.dot(h, W1[...], preferred_element_type=jnp.float32) + b1[...], 0.0)
    cur = jnp.dot(h, W2[...], preferred_element_type=jnp.float32) + b2[...] \
        + k[:, 0:2]
    cus = jnp.where(validc > 0.0, cur, _SENT)
    gt_o[0] = jnp.concatenate(
        [cus, dv[0], jnp.zeros((_RB, 125), jnp.float32)], axis=1)
    sig = _sigm(cus)
    pp_o[0] = jnp.concatenate([sig[:, 1:2], sig[:, 0:1]], axis=1)


def _dense(mem, kc, dvr, eW, eb, ng, nb, cW, cb, W0, b0, W1, b1, W2, b2):
    nrb = _S8 // _RB
    wspec = lambda shp: pl.BlockSpec(shp, lambda b, r: (0,) * len(shp))
    return _pallas_call(
        _dense_body,
        grid=(_B, nrb),
        in_specs=[
            pl.BlockSpec((1, _RB, _C), lambda b, r: (b, _S4 // _RB + r, 0)),
            pl.BlockSpec((_RB, 4), lambda b, r: (r, 0)),
            pl.BlockSpec((1, _RB, 1), lambda b, r: (b, r, 0)),
            wspec((_C, _C)), wspec((1, _C)), wspec((1, _C)), wspec((1, _C)),
            wspec((_C, 2)), wspec((1, 2)),
            wspec((_C, _C)), wspec((1, _C)),
            wspec((_C, _C)), wspec((1, _C)),
            wspec((_C, 2)), wspec((1, 2)),
        ],
        out_specs=[
            pl.BlockSpec((1, _RB, 2), lambda b, r: (b, r, 0)),
            pl.BlockSpec((1, _RB, 2), lambda b, r: (b, r, 0)),
            pl.BlockSpec((1, _RB, 128), lambda b, r: (b, r, 0)),
            pl.BlockSpec((1, _RB, 1), lambda b, r: (b, r, 0)),
        ],
        out_shape=[
            jax.ShapeDtypeStruct((_B, _S8, 2), jnp.float32),
            jax.ShapeDtypeStruct((_B, _S8, 2), jnp.float32),
            jax.ShapeDtypeStruct((_B, _S8, 128), jnp.float32),
            jax.ShapeDtypeStruct((_B, _S8, 1), jnp.float32),
        ],
    )(mem, kc, dvr, eW, eb, ng, nb, cW, cb, W0, b0, W1, b1, W2, b2)


def _sort_body(sc_ref, idx_o):
    i = lax.broadcasted_iota(jnp.int32, (_B, _S8), 1)
    key = sc_ref[...]
    idx = i
    kk = 2
    while kk <= _S8:
        asc = (i & kk) != 0
        j = kk // 2
        while j >= 1:
            upper = (i & j) != 0
            pkey = jnp.where(upper, _roll(key, j), _roll(key, _S8 - j))
            pidx = jnp.where(upper, _roll(idx, j), _roll(idx, _S8 - j))
            bp = (pkey > key) | ((pkey == key) & (pidx < idx))
            take = bp ^ upper ^ asc
            key = jnp.where(take, pkey, key)
            idx = jnp.where(take, pidx, idx)
            j //= 2
        kk *= 2
    idx_o[...] = idx


def _sort(scores):
    return _pallas_call(
        _sort_body,
        out_shape=jax.ShapeDtypeStruct((_B, _S8), jnp.int32),
    )(scores)


def _rtne(x):
    f = jnp.floor(x)
    d = x - f
    c = f + 1.0
    f_even = jnp.floor(f * 0.5) * 2.0 == f
    return jnp.where(d < 0.5, f, jnp.where(d > 0.5, c, jnp.where(f_even, f, c)))


def _query_body(qr, ptW, ptb, png, pnb, dimt, even,
                refp_o, qe_o, widx_o, wt_o, didx_o):
    b = pl.program_id(0)
    rows = qr[0]
    rx = _sigm(rows[:, 0:1])
    ry = _sigm(rows[:, 1:2])
    refp_o[0] = jnp.concatenate([rx, ry], axis=1)
    two_pi = jnp.float32(2.0 * np.pi)
    args = jnp.concatenate(
        [(ry * two_pi) / dimt[...], (rx * two_pi) / dimt[...]], axis=1)
    posq = jnp.where(even[...] > 0.0, jnp.sin(args), jnp.cos(args))
    q0 = jnp.dot(posq, ptW[...], preferred_element_type=jnp.float32) \
        + ptb[...]
    qe_o[0] = _layernorm(q0, png[...], pnb[...])
    gx = 2.0 * rx - 1.0
    gy = 2.0 * ry - 1.0
    ix = (gx + 1.0) * 64.0 / 2.0 - 0.5
    iy = (gy + 1.0) * 64.0 / 2.0 - 0.5
    ix0 = jnp.floor(ix)
    iy0 = jnp.floor(iy)
    ix1 = ix0 + 1.0
    iy1 = iy0 + 1.0
    wx1 = ix - ix0
    wx0 = 1.0 - wx1
    wy1 = iy - iy0
    wy0 = 1.0 - wy1
    boff = b * _S8

    def corner(iyc, ixc, wv):
        ok = (ixc >= 0.0) & (ixc <= 63.0) & (iyc >= 0.0) & (iyc <= 63.0)
        xi = jnp.clip(ixc, 0.0, 63.0).astype(jnp.int32)
        yi = jnp.clip(iyc, 0.0, 63.0).astype(jnp.int32)
        return yi * 64 + xi + boff, jnp.where(ok, wv, 0.0)

    i00, w00 = corner(iy0, ix0, wy0 * wx0)
    i01, w01 = corner(iy0, ix1, wy0 * wx1)
    i10, w10 = corner(iy1, ix0, wy1 * wx0)
    i11, w11 = corner(iy1, ix1, wy1 * wx1)
    widx_o[0] = jnp.concatenate([i00, i01, i10, i11], axis=1)
    wt_o[0] = jnp.concatenate([w00, w01, w10, w11], axis=1)
    rpx = _rtne(rx * 64.0)
    rpy = _rtne(ry * 64.0)
    dvi = jnp.clip(rpy * 64.0 + rpx, 0.0, 4095.0).astype(jnp.int32)
    didx_o[0] = dvi + boff


def _query(qrows, ptW, ptb, png, pnb, dimt, even):
    wspec = lambda shp: pl.BlockSpec(shp, lambda b: (0,) * len(shp))
    return _pallas_call(
        _query_body,
        grid=(_B,),
        in_specs=[
            pl.BlockSpec((1, _NQP, 128), lambda b: (b, 0, 0)),
            wspec((_C, _C)), wspec((1, _C)), wspec((1, _C)), wspec((1, _C)),
            wspec((1, 128)), wspec((1, _C)),
        ],
        out_specs=[
            pl.BlockSpec((1, _NQP, 2), lambda b: (b, 0, 0)),
            pl.BlockSpec((1, _NQP, _C), lambda b: (b, 0, 0)),
            pl.BlockSpec((1, _NQP, 4), lambda b: (b, 0, 0)),
            pl.BlockSpec((1, _NQP, 4), lambda b: (b, 0, 0)),
            pl.BlockSpec((1, _NQP, 1), lambda b: (b, 0, 0)),
        ],
        out_shape=[
            jax.ShapeDtypeStruct((_B, _NQP, 2), jnp.float32),
            jax.ShapeDtypeStruct((_B, _NQP, _C), jnp.float32),
            jax.ShapeDtypeStruct((_B, _NQP, 4), jnp.int32),
            jax.ShapeDtypeStruct((_B, _NQP, 4), jnp.float32),
            jax.ShapeDtypeStruct((_B, _NQP, 1), jnp.int32),
        ],
    )(qrows, ptW, ptb, png, pnb, dimt, even)


def _sc_gather1(gtab, qidx):
    ndiv = _B * _NQP
    dpw = ndiv // _NW

    mesh = plsc.VectorSubcoreMesh(core_axis_name="c", subcore_axis_name="s")

    @functools.partial(
        pl.kernel, mesh=mesh,
        out_type=jax.ShapeDtypeStruct((ndiv, 128), jnp.float32),
        scratch_types=[
            pltpu.VMEM((dpw,), jnp.int32),
            pltpu.VMEM((dpw, 128), jnp.float32),
            pltpu.SemaphoreType.DMA,
        ])
    def body(gt_h, qidx_h, out_h, idx_v, rows_v, sem):
        wid = lax.axis_index("s") * 2 + lax.axis_index("c")
        base = wid * dpw
        pltpu.sync_copy(qidx_h.at[pl.ds(base, dpw)], idx_v)
        pltpu.async_copy(gt_h.at[idx_v], rows_v, sem).wait()
        pltpu.sync_copy(rows_v, out_h.at[pl.ds(base, dpw)])

    return body(gtab, qidx)


def _sc_gather(bt, gidx, gtab, dgidx):
    nrows = _B * _NQP * 4
    rpw = nrows // _NW
    nch = rpw // _QCH
    ndiv = _B * _NQP
    dpw = ndiv // _NW

    mesh = plsc.VectorSubcoreMesh(core_axis_name="c", subcore_axis_name="s")

    @functools.partial(
        pl.kernel, mesh=mesh,
        out_type=(jax.ShapeDtypeStruct((nrows, _C), jnp.float32),
                  jax.ShapeDtypeStruct((ndiv, 128), jnp.float32)),
        scratch_types=[
            pltpu.VMEM((_QCH,), jnp.int32),
            pltpu.VMEM((_QCH, _C), jnp.float32),
            pltpu.VMEM((dpw,), jnp.int32),
            pltpu.VMEM((dpw, 128), jnp.float32),
            pltpu.SemaphoreType.DMA,
        ])
    def body(bt_h, gidx_h, gt_h, dgidx_h, out_h, dnr_h,
             idx_v, rows_v, didx_v, drow_v, sem):
        wid = lax.axis_index("s") * 2 + lax.axis_index("c")
        base = wid * rpw
        for k in range(nch):
            o = base + k * _QCH
            pltpu.sync_copy(gidx_h.at[pl.ds(o, _QCH)], idx_v)
            pltpu.async_copy(bt_h.at[idx_v], rows_v, sem).wait()
            pltpu.sync_copy(rows_v, out_h.at[pl.ds(o, _QCH)])
        dbase = wid * dpw
        pltpu.sync_copy(dgidx_h.at[pl.ds(dbase, dpw)], didx_v)
        pltpu.async_copy(gt_h.at[didx_v], drow_v, sem).wait()
        pltpu.sync_copy(drow_v, dnr_h.at[pl.ds(dbase, dpw)])

    return body(bt, gidx, gtab, dgidx)


def _combine_body(co, wt, qf_o):
    x = co[0]
    w = wt[0]
    acc = x[0:_NQP] * w[:, 0:1]
    for c in range(1, 4):
        acc = acc + x[c * _NQP:(c + 1) * _NQP] * w[:, c:c + 1]
    qf_o[0] = acc


def _combine(corners, wts):
    return _pallas_call(
        _combine_body,
        grid=(_B,),
        in_specs=[
            pl.BlockSpec((1, 4 * _NQP, _C), lambda b: (b, 0, 0)),
            pl.BlockSpec((1, _NQP, 4), lambda b: (b, 0, 0)),
        ],
        out_specs=pl.BlockSpec((1, _NQP, _C), lambda b: (b, 0, 0)),
        out_shape=jax.ShapeDtypeStruct((_B, _NQP, _C), jnp.float32),
    )(corners, wts)


def kernel(memory, mask_flatten, backbone_features, div,
           enc_output_W, enc_output_b, enc_norm_g, enc_norm_b,
           cls_W, cls_b, mlp_W0, mlp_b0, mlp_W1, mlp_b1, mlp_W2, mlp_b2,
           pt_W, pt_b, ptn_g, ptn_b):
    f32 = jnp.float32
    kc, dimt, even = _const_tables()
    r1 = lambda v: v.reshape(1, -1).astype(f32)
    cls_out, pp, gtab, sc3 = _dense(
        memory, kc, div.reshape(_B, _S8, 1), enc_output_W, r1(enc_output_b),
        r1(enc_norm_g), r1(enc_norm_b), cls_W, r1(cls_b), mlp_W0, r1(mlp_b0),
        mlp_W1, r1(mlp_b1), mlp_W2, r1(mlp_b2))
    sidx = _sort(sc3.reshape(_B, _S8))
    gt_flat = gtab.reshape(_B * _S8, 128)
    qidx = (sidx[:, :_NQP]
            + (jnp.arange(_B, dtype=jnp.int32) * _S8)[:, None]).reshape(-1)
    qrows = _sc_gather1(gt_flat, qidx)
    refp, qe, widx, wt, didx = _query(
        qrows.reshape(_B, _NQP, 128), pt_W, r1(pt_b), r1(ptn_g), r1(ptn_b),
        dimt, even)
    bt = backbone_features.reshape(_B, _C, _S8).transpose(0, 2, 1) \
        .reshape(_B * _S8, _C)
    corners, dnr = _sc_gather(bt, widx.transpose(0, 2, 1).reshape(-1),
                              gt_flat, didx.reshape(-1))
    qf = _combine(corners.reshape(_B, 4 * _NQP, _C), wt)
    return (cls_out, pp, dnr[:, 2].reshape(_B, _NQP)[:, :_NQ],
            qf[:, :_NQ], qe[:, :_NQ], refp[:, :_NQ])
